# baseline (speedup 1.0000x reference)
import jax
import jax.numpy as jnp
from jax import lax
from jax.experimental import pallas as pl
from jax.experimental.pallas import tpu as pltpu
from jax.experimental.pallas import tpu_sc as plsc

EMBED_DIM = 64
NUM_PACKETS = 130560
CHUNK = 510
SEQ = CHUNK + 2
NUM_CHUNKS = NUM_PACKETS // CHUNK

_NC = 2
_NS = 16
_NW = _NC * _NS
_CPW = NUM_CHUNKS // _NW
_LANES = 16
_HCOL = SEQ // 2
_HBLK = _HCOL // _LANES
_XW = 384
_AMAX = NUM_PACKETS - _XW
_DW = 5120


def _body(x_hbm, dir_hbm, tok_hbm, dtab_hbm, pos_hbm, out_hbm,
          xv0, xv1, ov0, ov1, pos_v, tok_v, dtab_v, d_all, dtab_sm,
          sx0, sx1, so0, so1):
    wid = lax.axis_index("c") * _NS + lax.axis_index("s")

    pltpu.sync_copy(tok_hbm, tok_v)
    pltpu.sync_copy(dtab_hbm, dtab_v)
    pltpu.sync_copy(pos_hbm, pos_v)
    dstart = pl.multiple_of((wid * _CPW * CHUNK // 1024) * 1024, 1024)
    doff = wid * _CPW * CHUNK - dstart
    pltpu.sync_copy(dir_hbm.at[pl.ds(dstart, _DW)], d_all)

    for r in range(3):
        for t in range(4):
            v = dtab_v[r, pl.ds(16 * t, _LANES)]
            for l in range(_LANES):
                dtab_sm[r, 16 * t + l] = v[l]

    lane = lax.iota(jnp.int32, _LANES)

    def full(v):
        return jnp.full((_LANES,), v, jnp.int32)

    edge0 = []
    edge1 = []
    for t in range(4):
        rows = lane + (16 * t)
        clsf = plsc.load_gather(tok_v, [full(1), rows])
        sep = plsc.load_gather(tok_v, [full(2), rows])
        p0 = plsc.load_gather(pos_v, [rows, full(0)])
        p1 = plsc.load_gather(pos_v, [rows, full(SEQ - 1)])
        edge0.append(clsf * 2.0 + p0)
        edge1.append(sep * 2.0 + p1)

    def x_window(cc, h):
        xs = cc * CHUNK + (_HCOL - 1) * h
        a = pl.multiple_of(jnp.minimum((xs // 128) * 128, _AMAX), 128)
        return a, xs - a

    def start_x(cc, h, xv, sem):
        a, _ = x_window(cc, h)
        pltpu.async_copy(x_hbm.at[0, :, pl.ds(a, _XW)], xv, sem)

    def wait_x(xv, sem):
        pltpu.make_async_copy(x_hbm.at[0, :, pl.ds(0, _XW)], xv, sem).wait()

    def wait_out(ov, sem):
        pltpu.make_async_copy(ov, out_hbm.at[0, :, pl.ds(0, _HCOL)],
                              sem).wait()

    def compute_half(kk, cc, h, xv, ov):
        _, off = x_window(cc, h)
        dbase = doff + kk * CHUNK

        def col_block(kb, c2):
            colg = lane + (_HCOL * h) + kb * _LANES
            didx = jnp.clip(colg + (dbase - 1), 0, _DW - 1)
            d16 = plsc.load_gather(d_all, [didx])
            dcl = jnp.clip(d16, 0, 2)
            is1 = dcl == 1
            is2 = dcl == 2
            xidx = jnp.clip(colg + (off - (_HCOL - 1) * h - 1), 0, _XW - 1)
            co = pl.multiple_of(kb * _LANES, _LANES)
            po = pl.multiple_of(_HCOL * h + kb * _LANES, _LANES)

            def emb_row(e, c3):
                xvv = plsc.load_gather(xv, [full(e), xidx])
                pv = pos_v[e, pl.ds(po, _LANES)]
                s0 = dtab_sm[0, e]
                s1 = dtab_sm[1, e]
                s2 = dtab_sm[2, e]
                dg = jnp.where(is2, s2, jnp.where(is1, s1, s0))
                ov[e, pl.ds(co, _LANES)] = xvv + pv + dg
                return c3

            lax.fori_loop(0, EMBED_DIM, emb_row, 0, unroll=EMBED_DIM)
            return c2

        lax.fori_loop(0, _HBLK, col_block, 0, unroll=2)

        for t in range(4):
            rows = lane + (16 * t)
            if h == 0:
                plsc.store_scatter(ov, [rows, full(0)], edge0[t])
            else:
                plsc.store_scatter(ov, [rows, full(_HCOL - 1)], edge1[t])

    start_x(wid * _CPW, 0, xv0, sx0)

    def do_chunk(k, carry):
        cc = wid * _CPW + k

        start_x(cc, 1, xv1, sx1)

        wait_x(xv0, sx0)

        @pl.when(k >= 1)
        def _():
            wait_out(ov0, so0)

        compute_half(k, cc, 0, xv0, ov0)
        pltpu.async_copy(ov0, out_hbm.at[cc, :, pl.ds(0, _HCOL)], so0)

        @pl.when(k <= _CPW - 2)
        def _():
            start_x(cc + 1, 0, xv0, sx0)

        wait_x(xv1, sx1)

        @pl.when(k >= 1)
        def _():
            wait_out(ov1, so1)

        compute_half(k, cc, 1, xv1, ov1)
        pltpu.async_copy(ov1, out_hbm.at[cc, :, pl.ds(_HCOL, _HCOL)], so1)
        return carry

    lax.fori_loop(0, _CPW, do_chunk, 0)
    wait_out(ov0, so0)
    wait_out(ov1, so1)


@jax.jit
def _flow_embed(xT, dirv, tok, dtab, posT):
    mesh = plsc.VectorSubcoreMesh(core_axis_name="c", subcore_axis_name="s")
    f = pl.kernel(
        _body,
        out_type=jax.ShapeDtypeStruct((NUM_CHUNKS, EMBED_DIM, SEQ), jnp.float32),
        mesh=mesh,
        scratch_types=[
            pltpu.VMEM((EMBED_DIM, _XW), jnp.float32),
            pltpu.VMEM((EMBED_DIM, _XW), jnp.float32),
            pltpu.VMEM((EMBED_DIM, _HCOL), jnp.float32),
            pltpu.VMEM((EMBED_DIM, _HCOL), jnp.float32),
            pltpu.VMEM((EMBED_DIM, SEQ), jnp.float32),
            pltpu.VMEM((5, EMBED_DIM), jnp.float32),
            pltpu.VMEM((3, EMBED_DIM), jnp.float32),
            pltpu.VMEM((_DW,), jnp.int32),
            pltpu.SMEM((3, EMBED_DIM), jnp.float32),
            pltpu.SemaphoreType.DMA,
            pltpu.SemaphoreType.DMA,
            pltpu.SemaphoreType.DMA,
            pltpu.SemaphoreType.DMA,
        ],
        compiler_params=pltpu.CompilerParams(use_tc_tiling_on_sc=True,
                                             needs_layout_passes=False),
    )
    return f(xT, dirv, tok, dtab, posT)


def kernel(cls_packet_embeddings, direction, token_embed, direction_embed,
           packet_pos_embed):
    xT = cls_packet_embeddings.transpose(1, 2, 0)
    posT = packet_pos_embed.transpose(1, 0)
    raw = _flow_embed(xT, direction.astype(jnp.int32), token_embed,
                      direction_embed, posT)
    embed_val = raw.transpose(0, 2, 1)
    pad_indices = jnp.zeros((NUM_CHUNKS, SEQ), dtype=bool)
    pad_indices = pad_indices.at[:, 0].set(True).at[:, -1].set(True)
    return (embed_val, pad_indices)


# speedup vs baseline: 1.4016x; 1.4016x over previous
import jax
import jax.numpy as jnp
from jax import lax
from jax.experimental import pallas as pl
from jax.experimental.pallas import tpu as pltpu
from jax.experimental.pallas import tpu_sc as plsc

EMBED_DIM = 64
NUM_PACKETS = 130560
CHUNK = 510
SEQ = CHUNK + 2
NUM_CHUNKS = NUM_PACKETS // CHUNK

_NC = 2
_NS = 16
_NW = _NC * _NS
_CPW = NUM_CHUNKS // _NW
_LANES = 16
_HCOL = SEQ // 2
_HBLK = _HCOL // _LANES
_XW = 384
_AMAX = NUM_PACKETS - _XW
_DW = 5120


def _body(x_hbm, dir_hbm, tok_hbm, dtab_hbm, pos_hbm, out_hbm,
          xv0, xv1, ov0, ov1, pos_v, tok_v, dtab_v, d_all, dtab_sm,
          sx0, sx1, so0, so1):
    wid = lax.axis_index("c") * _NS + lax.axis_index("s")

    pltpu.sync_copy(tok_hbm, tok_v)
    pltpu.sync_copy(dtab_hbm, dtab_v)
    pltpu.sync_copy(pos_hbm, pos_v)
    dstart = pl.multiple_of((wid * _CPW * CHUNK // 1024) * 1024, 1024)
    doff = wid * _CPW * CHUNK - dstart
    pltpu.sync_copy(dir_hbm.at[pl.ds(dstart, _DW)], d_all)

    for r in range(3):
        for t in range(4):
            v = dtab_v[r, pl.ds(16 * t, _LANES)]
            for l in range(_LANES):
                dtab_sm[r, 16 * t + l] = v[l]

    lane = lax.iota(jnp.int32, _LANES)

    def full(v):
        return jnp.full((_LANES,), v, jnp.int32)

    edge0 = []
    edge1 = []
    for t in range(4):
        rows = lane + (16 * t)
        clsf = plsc.load_gather(tok_v, [full(1), rows])
        sep = plsc.load_gather(tok_v, [full(2), rows])
        p0 = plsc.load_gather(pos_v, [rows, full(0)])
        p1 = plsc.load_gather(pos_v, [rows, full(SEQ - 1)])
        edge0.append(clsf * 2.0 + p0)
        edge1.append(sep * 2.0 + p1)

    def x_window(cc, h):
        xs = cc * CHUNK + (_HCOL - 1) * h
        a = pl.multiple_of(jnp.minimum((xs // 128) * 128, _AMAX), 128)
        return a, xs - a

    def start_x(cc, h, xv, sem):
        a, _ = x_window(cc, h)
        pltpu.async_copy(x_hbm.at[0, :, pl.ds(a, _XW)], xv, sem)

    def wait_x(xv, sem):
        pltpu.make_async_copy(x_hbm.at[0, :, pl.ds(0, _XW)], xv, sem).wait()

    def wait_out(ov, sem):
        pltpu.make_async_copy(ov, out_hbm.at[0, :, pl.ds(0, _HCOL)],
                              sem).wait()

    def compute_half(kk, cc, h, xv, ov):
        _, off = x_window(cc, h)
        dbase = doff + kk * CHUNK

        def col_block(kb, c2):
            colg = lane + (_HCOL * h) + kb * _LANES
            didx = jnp.clip(colg + (dbase - 1), 0, _DW - 1)
            d16 = plsc.load_gather(d_all, [didx])
            dcl = jnp.clip(d16, 0, 2)
            is1 = dcl == 1
            is2 = dcl == 2
            xidx = jnp.clip(colg + (off - (_HCOL - 1) * h - 1), 0, _XW - 1)
            co = pl.multiple_of(kb * _LANES, _LANES)
            po = pl.multiple_of(_HCOL * h + kb * _LANES, _LANES)

            def emb_row(e, c3):
                xvv = plsc.load_gather(xv, [full(e), xidx])
                pv = pos_v[e, pl.ds(po, _LANES)]
                s0 = dtab_sm[0, e]
                s1 = dtab_sm[1, e]
                s2 = dtab_sm[2, e]
                dg = jnp.where(is2, s2, jnp.where(is1, s1, s0))
                ov[e, pl.ds(co, _LANES)] = xvv + pv + dg
                return c3

            lax.fori_loop(0, EMBED_DIM, emb_row, 0, unroll=EMBED_DIM)
            return c2

        lax.fori_loop(0, _HBLK, col_block, 0)

        for t in range(4):
            rows = lane + (16 * t)
            if h == 0:
                plsc.store_scatter(ov, [rows, full(0)], edge0[t])
            else:
                plsc.store_scatter(ov, [rows, full(_HCOL - 1)], edge1[t])

    start_x(wid * _CPW, 0, xv0, sx0)

    def do_chunk(k, carry):
        cc = wid * _CPW + k

        start_x(cc, 1, xv1, sx1)

        wait_x(xv0, sx0)

        @pl.when(k >= 1)
        def _():
            wait_out(ov0, so0)

        compute_half(k, cc, 0, xv0, ov0)
        pltpu.async_copy(ov0, out_hbm.at[cc, :, pl.ds(0, _HCOL)], so0)

        @pl.when(k <= _CPW - 2)
        def _():
            start_x(cc + 1, 0, xv0, sx0)

        wait_x(xv1, sx1)

        @pl.when(k >= 1)
        def _():
            wait_out(ov1, so1)

        compute_half(k, cc, 1, xv1, ov1)
        pltpu.async_copy(ov1, out_hbm.at[cc, :, pl.ds(_HCOL, _HCOL)], so1)
        return carry

    lax.fori_loop(0, _CPW, do_chunk, 0)
    wait_out(ov0, so0)
    wait_out(ov1, so1)


@jax.jit
def _flow_embed(xT, dirv, tok, dtab, posT):
    mesh = plsc.VectorSubcoreMesh(core_axis_name="c", subcore_axis_name="s")
    f = pl.kernel(
        _body,
        out_type=jax.ShapeDtypeStruct((NUM_CHUNKS, EMBED_DIM, SEQ), jnp.float32),
        mesh=mesh,
        scratch_types=[
            pltpu.VMEM((EMBED_DIM, _XW), jnp.float32),
            pltpu.VMEM((EMBED_DIM, _XW), jnp.float32),
            pltpu.VMEM((EMBED_DIM, _HCOL), jnp.float32),
            pltpu.VMEM((EMBED_DIM, _HCOL), jnp.float32),
            pltpu.VMEM((EMBED_DIM, SEQ), jnp.float32),
            pltpu.VMEM((5, EMBED_DIM), jnp.float32),
            pltpu.VMEM((3, EMBED_DIM), jnp.float32),
            pltpu.VMEM((_DW,), jnp.int32),
            pltpu.SMEM((3, EMBED_DIM), jnp.float32),
            pltpu.SemaphoreType.DMA,
            pltpu.SemaphoreType.DMA,
            pltpu.SemaphoreType.DMA,
            pltpu.SemaphoreType.DMA,
        ],
        compiler_params=pltpu.CompilerParams(use_tc_tiling_on_sc=True,
                                             needs_layout_passes=False),
    )
    return f(xT, dirv, tok, dtab, posT)


def kernel(cls_packet_embeddings, direction, token_embed, direction_embed,
           packet_pos_embed):
    xT = cls_packet_embeddings.transpose(1, 2, 0)
    posT = packet_pos_embed.transpose(1, 0)
    raw = _flow_embed(xT, direction.astype(jnp.int32), token_embed,
                      direction_embed, posT)
    embed_val = raw.transpose(0, 2, 1)
    pad_indices = jnp.zeros((NUM_CHUNKS, SEQ), dtype=bool)
    pad_indices = pad_indices.at[:, 0].set(True).at[:, -1].set(True)
    return (embed_val, pad_indices)
